# Initial kernel scaffold; baseline (speedup 1.0000x reference)
import jax
import jax.numpy as jnp
from jax.experimental import pallas as pl


def kernel(x_drug, x_se, ei_interacts, ei_causes, ei_caused_by, W1_rel_ii, b1_rel_ii, W1_root_ii, W1_rel_c, b1_rel_c, W1_root_c, W1_rel_cb, b1_rel_cb, W1_root_cb, W2_rel_ii, b2_rel_ii, W2_root_ii, W2_rel_c, b2_rel_c, W2_root_c, W2_rel_cb, b2_rel_cb, W2_root_cb, bn_gamma, bn_beta):
    raise NotImplementedError("write your pallas kernel here")


# speedup vs baseline: 3.7319x; 3.7319x over previous
import functools

import jax
import jax.numpy as jnp
from jax import lax
from jax.experimental import pallas as pl
from jax.experimental.pallas import tpu as pltpu
from jax.experimental.pallas import tpu_sc as plsc

ND = 10000
D = 128
E = 320000
NC = 2
NS = 16
NW = NC * NS
EPW = E // NW
K = 80
NCH = EPW // K
RPT = 624
TAIL = ND - NS * RPT
ZR = 208

_f32 = jnp.float32


def _seg3_body(xa, xb, si_ii, di_ii, si_c, di_c, si_cb, di_cb,
               out_ii, out_c, out_cb,
               acc, zbuf, sidx, didx, rows, sem):
    c = lax.axis_index("c")
    s = lax.axis_index("s")
    wid = c * NS + s

    z16 = jnp.zeros((16,), _f32)

    def zrow(r, carry):
        for j in range(D // 16):
            zbuf[r, pl.ds(j * 16, 16)] = z16
        return carry

    lax.fori_loop(0, ZR, zrow, 0)

    for x_hbm, si_hbm, di_hbm, out_hbm in (
        (xa, si_ii, di_ii, out_ii),
        (xa, si_c, di_c, out_c),
        (xb, si_cb, di_cb, out_cb),
    ):
        for j in range(RPT // ZR):
            pltpu.sync_copy(zbuf, acc.at[pl.ds(s * RPT + j * ZR, ZR)])

        @pl.when(s == NS - 1)
        def _():
            pltpu.sync_copy(zbuf.at[pl.ds(0, TAIL)],
                            acc.at[pl.ds(NS * RPT, TAIL)])

        plsc.subcore_barrier()

        base = wid * EPW

        def chunk(i, carry):
            off = pl.multiple_of(base + i * K, 8)
            pltpu.sync_copy(si_hbm.at[pl.ds(off, K)], sidx)
            pltpu.sync_copy(di_hbm.at[pl.ds(off, K)], didx)
            pltpu.async_copy(x_hbm.at[sidx], rows, sem).wait()
            pltpu.sync_copy(rows, acc.at[didx], add=True)
            return carry

        lax.fori_loop(0, NCH, chunk, 0)
        plsc.subcore_barrier()

        pltpu.sync_copy(acc.at[pl.ds(s * RPT, RPT)],
                        out_hbm.at[c, pl.ds(s * RPT, RPT)])

        @pl.when(s == NS - 1)
        def _():
            pltpu.sync_copy(acc.at[pl.ds(NS * RPT, TAIL)],
                            out_hbm.at[c, pl.ds(NS * RPT, TAIL)])


def _seg3(xa, xb, si_ii, di_ii, si_c, di_c, si_cb, di_cb):
    mesh = plsc.VectorSubcoreMesh(core_axis_name="c", subcore_axis_name="s")
    f = functools.partial(
        pl.kernel,
        mesh=mesh,
        out_type=[jax.ShapeDtypeStruct((NC, ND, D), _f32)] * 3,
        scratch_types=[
            pltpu.VMEM_SHARED((ND, D), _f32),
            pltpu.VMEM((ZR, D), _f32),
            pltpu.VMEM((K,), jnp.int32),
            pltpu.VMEM((K,), jnp.int32),
            pltpu.VMEM((K, D), _f32),
            pltpu.SemaphoreType.DMA,
        ],
    )(_seg3_body)
    return f(xa, xb, si_ii, di_ii, si_c, di_c, si_cb, di_cb)


_RB = 1000


def _dense_body(aii, acb, ac, xd, xs, wri, wrcb, wrc, wrd, wrs, bd, bs,
                d_o, s_o):
    agg_ii = aii[0] + aii[1]
    agg_cb = acb[0] + acb[1]
    agg_c = ac[0] + ac[1]
    d = (jnp.dot(agg_ii, wri[...], preferred_element_type=_f32)
         + jnp.dot(agg_cb, wrcb[...], preferred_element_type=_f32)
         + jnp.dot(xd[...], wrd[...], preferred_element_type=_f32)
         + bd[...])
    s = (jnp.dot(agg_c, wrc[...], preferred_element_type=_f32)
         + jnp.dot(xs[...], wrs[...], preferred_element_type=_f32)
         + bs[...])
    d_o[...] = jnp.where(d >= 0, d, 0.01 * d)
    s_o[...] = jnp.where(s >= 0, s, 0.01 * s)


def _dense(p_ii, p_cb, p_c, xd, xs, wri, wrcb, wrc, wrd, wrs, bd, bs):
    n = ND // _RB
    part = pl.BlockSpec((2, _RB, D), lambda i: (0, i, 0))
    row = pl.BlockSpec((_RB, D), lambda i: (i, 0))
    mat = pl.BlockSpec((D, D), lambda i: (0, 0))
    vec = pl.BlockSpec((1, D), lambda i: (0, 0))
    return pl.pallas_call(
        _dense_body,
        grid=(n,),
        in_specs=[part, part, part, row, row, mat, mat, mat, mat, mat,
                  vec, vec],
        out_specs=[row, row],
        out_shape=[jax.ShapeDtypeStruct((ND, D), _f32)] * 2,
    )(p_ii, p_cb, p_c, xd, xs, wri, wrcb, wrc, wrd, wrs, bd, bs)


def _bn_body(d2, s2, g, b, d_o, s_o):
    gv = g[...]
    bv = b[...]
    for x, o in ((d2, d_o), (s2, s_o)):
        xv = x[...]
        m = jnp.mean(xv, axis=0, keepdims=True)
        cv = xv - m
        v = jnp.mean(cv * cv, axis=0, keepdims=True)
        o[...] = cv * lax.rsqrt(v + 1e-5) * gv + bv


def _bn(d2, s2, gamma, beta):
    full = pl.BlockSpec((ND, D), lambda: (0, 0))
    vec = pl.BlockSpec((1, D), lambda: (0, 0))
    return pl.pallas_call(
        _bn_body,
        in_specs=[full, full, vec, vec],
        out_specs=[full, full],
        out_shape=[jax.ShapeDtypeStruct((ND, D), _f32)] * 2,
    )(d2, s2, gamma, beta)


def kernel(x_drug, x_se, ei_interacts, ei_causes, ei_caused_by,
           W1_rel_ii, b1_rel_ii, W1_root_ii, W1_rel_c, b1_rel_c, W1_root_c,
           W1_rel_cb, b1_rel_cb, W1_root_cb,
           W2_rel_ii, b2_rel_ii, W2_root_ii, W2_rel_c, b2_rel_c, W2_root_c,
           W2_rel_cb, b2_rel_cb, W2_root_cb, bn_gamma, bn_beta):
    si_ii, di_ii = ei_interacts[0], ei_interacts[1]
    si_c, di_c = ei_causes[0], ei_causes[1]
    si_cb, di_cb = ei_caused_by[0], ei_caused_by[1]

    w1rd = W1_root_ii + W1_root_cb
    b1d = (b1_rel_ii + b1_rel_cb).reshape(1, D)
    b1s = b1_rel_c.reshape(1, D)
    w2rd = W2_root_ii + W2_root_cb
    b2d = (b2_rel_ii + b2_rel_cb).reshape(1, D)
    b2s = b2_rel_c.reshape(1, D)

    p_ii, p_c, p_cb = _seg3(x_drug, x_se, si_ii, di_ii, si_c, di_c,
                            si_cb, di_cb)
    d1, s1 = _dense(p_ii, p_cb, p_c, x_drug, x_se,
                    W1_rel_ii, W1_rel_cb, W1_rel_c, w1rd, W1_root_c,
                    b1d, b1s)
    q_ii, q_c, q_cb = _seg3(d1, s1, si_ii, di_ii, si_c, di_c, si_cb, di_cb)
    d2, s2 = _dense(q_ii, q_cb, q_c, d1, s1,
                    W2_rel_ii, W2_rel_cb, W2_rel_c, w2rd, W2_root_c,
                    b2d, b2s)
    return _bn(d2, s2, bn_gamma.reshape(1, D), bn_beta.reshape(1, D))


# speedup vs baseline: 7.1556x; 1.9174x over previous
import functools

import jax
import jax.numpy as jnp
from jax import lax
from jax.experimental import pallas as pl
from jax.experimental.pallas import tpu as pltpu
from jax.experimental.pallas import tpu_sc as plsc

ND = 10000
D = 128
E = 320000
NC = 2
NS = 16
NW = NC * NS
EPW = E // NW
K = 80
NCH = EPW // K
RPT = 624
TAIL = ND - NS * RPT
ZR = 208

_f32 = jnp.float32


def _seg3_body(xa, xb, si_ii, di_ii, si_c, di_c, si_cb, di_cb,
               out_ii, out_c, out_cb,
               acc, zbuf, sidx0, didx0, sidx1, didx1, rows0, rows1,
               smi0, smd0, smi1, smd1, smg0, smg1):
    c = lax.axis_index("c")
    s = lax.axis_index("s")
    wid = c * NS + s
    sbuf = (sidx0, sidx1)
    dbuf = (didx0, didx1)
    rbuf = (rows0, rows1)
    smi = (smi0, smi1)
    smd = (smd0, smd1)
    smg = (smg0, smg1)

    z16 = jnp.zeros((16,), _f32)

    def zrow(r, carry):
        for j in range(D // 16):
            zbuf[r, pl.ds(j * 16, 16)] = z16
        return carry

    lax.fori_loop(0, ZR, zrow, 0)

    for x_hbm, si_hbm, di_hbm, out_hbm in (
        (xa, si_ii, di_ii, out_ii),
        (xa, si_c, di_c, out_c),
        (xb, si_cb, di_cb, out_cb),
    ):
        for j in range(RPT // ZR):
            pltpu.sync_copy(zbuf, acc.at[pl.ds(s * RPT + j * ZR, ZR)])

        @pl.when(s == NS - 1)
        def _():
            pltpu.sync_copy(zbuf.at[pl.ds(0, TAIL)],
                            acc.at[pl.ds(NS * RPT, TAIL)])

        plsc.subcore_barrier()

        base = wid * EPW

        def idx_start(i, b):
            off = pl.multiple_of(jnp.minimum(base + i * K, E - K), 8)
            pltpu.async_copy(si_hbm.at[pl.ds(off, K)], sbuf[b], smi[b])
            pltpu.async_copy(di_hbm.at[pl.ds(off, K)], dbuf[b], smd[b])

        def idx_wait(b):
            pltpu.make_async_copy(si_hbm.at[pl.ds(0, K)], sbuf[b],
                                  smi[b]).wait()
            pltpu.make_async_copy(di_hbm.at[pl.ds(0, K)], dbuf[b],
                                  smd[b]).wait()

        def gather_start(x_hbm_, b):
            pltpu.async_copy(x_hbm_.at[sbuf[b]], rbuf[b], smg[b])

        def gather_wait(x_hbm_, b):
            pltpu.make_async_copy(x_hbm_.at[pl.ds(0, K)], rbuf[b],
                                  smg[b]).wait()

        idx_start(0, 0)
        idx_start(1, 1)
        idx_wait(0)
        gather_start(x_hbm, 0)

        def pair(j, carry):
            idx_wait(1)
            gather_wait(x_hbm, 0)
            gather_start(x_hbm, 1)
            pltpu.sync_copy(rbuf[0], acc.at[dbuf[0]], add=True)
            idx_start(2 * j + 2, 0)
            idx_wait(0)
            gather_wait(x_hbm, 1)
            gather_start(x_hbm, 0)
            pltpu.sync_copy(rbuf[1], acc.at[dbuf[1]], add=True)
            idx_start(2 * j + 3, 1)
            return carry

        lax.fori_loop(0, (NCH - 1) // 2, pair, 0)
        gather_wait(x_hbm, 0)
        pltpu.sync_copy(rbuf[0], acc.at[dbuf[0]], add=True)
        idx_wait(1)
        plsc.subcore_barrier()

        pltpu.sync_copy(acc.at[pl.ds(s * RPT, RPT)],
                        out_hbm.at[c, pl.ds(s * RPT, RPT)])

        @pl.when(s == NS - 1)
        def _():
            pltpu.sync_copy(acc.at[pl.ds(NS * RPT, TAIL)],
                            out_hbm.at[c, pl.ds(NS * RPT, TAIL)])


def _seg3(xa, xb, si_ii, di_ii, si_c, di_c, si_cb, di_cb):
    mesh = plsc.VectorSubcoreMesh(core_axis_name="c", subcore_axis_name="s")
    f = functools.partial(
        pl.kernel,
        mesh=mesh,
        out_type=[jax.ShapeDtypeStruct((NC, ND, D), _f32)] * 3,
        scratch_types=[
            pltpu.VMEM_SHARED((ND, D), _f32),
            pltpu.VMEM((ZR, D), _f32),
            pltpu.VMEM((K,), jnp.int32),
            pltpu.VMEM((K,), jnp.int32),
            pltpu.VMEM((K,), jnp.int32),
            pltpu.VMEM((K,), jnp.int32),
            pltpu.VMEM((K, D), _f32),
            pltpu.VMEM((K, D), _f32),
            pltpu.SemaphoreType.DMA,
            pltpu.SemaphoreType.DMA,
            pltpu.SemaphoreType.DMA,
            pltpu.SemaphoreType.DMA,
            pltpu.SemaphoreType.DMA,
            pltpu.SemaphoreType.DMA,
        ],
    )(_seg3_body)
    return f(xa, xb, si_ii, di_ii, si_c, di_c, si_cb, di_cb)


_RB = 1000


def _dense_body(aii, acb, ac, xd, xs, wri, wrcb, wrc, wrd, wrs, bd, bs,
                d_o, s_o):
    agg_ii = aii[0] + aii[1]
    agg_cb = acb[0] + acb[1]
    agg_c = ac[0] + ac[1]
    d = (jnp.dot(agg_ii, wri[...], preferred_element_type=_f32)
         + jnp.dot(agg_cb, wrcb[...], preferred_element_type=_f32)
         + jnp.dot(xd[...], wrd[...], preferred_element_type=_f32)
         + bd[...])
    s = (jnp.dot(agg_c, wrc[...], preferred_element_type=_f32)
         + jnp.dot(xs[...], wrs[...], preferred_element_type=_f32)
         + bs[...])
    d_o[...] = jnp.where(d >= 0, d, 0.01 * d)
    s_o[...] = jnp.where(s >= 0, s, 0.01 * s)


def _dense(p_ii, p_cb, p_c, xd, xs, wri, wrcb, wrc, wrd, wrs, bd, bs):
    n = ND // _RB
    part = pl.BlockSpec((2, _RB, D), lambda i: (0, i, 0))
    row = pl.BlockSpec((_RB, D), lambda i: (i, 0))
    mat = pl.BlockSpec((D, D), lambda i: (0, 0))
    vec = pl.BlockSpec((1, D), lambda i: (0, 0))
    return pl.pallas_call(
        _dense_body,
        grid=(n,),
        in_specs=[part, part, part, row, row, mat, mat, mat, mat, mat,
                  vec, vec],
        out_specs=[row, row],
        out_shape=[jax.ShapeDtypeStruct((ND, D), _f32)] * 2,
    )(p_ii, p_cb, p_c, xd, xs, wri, wrcb, wrc, wrd, wrs, bd, bs)


def _bn_body(d2, s2, g, b, d_o, s_o):
    gv = g[...]
    bv = b[...]
    for x, o in ((d2, d_o), (s2, s_o)):
        xv = x[...]
        m = jnp.mean(xv, axis=0, keepdims=True)
        cv = xv - m
        v = jnp.mean(cv * cv, axis=0, keepdims=True)
        o[...] = cv * lax.rsqrt(v + 1e-5) * gv + bv


def _bn(d2, s2, gamma, beta):
    full = pl.BlockSpec((ND, D), lambda: (0, 0))
    vec = pl.BlockSpec((1, D), lambda: (0, 0))
    return pl.pallas_call(
        _bn_body,
        in_specs=[full, full, vec, vec],
        out_specs=[full, full],
        out_shape=[jax.ShapeDtypeStruct((ND, D), _f32)] * 2,
    )(d2, s2, gamma, beta)


def kernel(x_drug, x_se, ei_interacts, ei_causes, ei_caused_by,
           W1_rel_ii, b1_rel_ii, W1_root_ii, W1_rel_c, b1_rel_c, W1_root_c,
           W1_rel_cb, b1_rel_cb, W1_root_cb,
           W2_rel_ii, b2_rel_ii, W2_root_ii, W2_rel_c, b2_rel_c, W2_root_c,
           W2_rel_cb, b2_rel_cb, W2_root_cb, bn_gamma, bn_beta):
    si_ii, di_ii = ei_interacts[0], ei_interacts[1]
    si_c, di_c = ei_causes[0], ei_causes[1]
    si_cb, di_cb = ei_caused_by[0], ei_caused_by[1]

    w1rd = W1_root_ii + W1_root_cb
    b1d = (b1_rel_ii + b1_rel_cb).reshape(1, D)
    b1s = b1_rel_c.reshape(1, D)
    w2rd = W2_root_ii + W2_root_cb
    b2d = (b2_rel_ii + b2_rel_cb).reshape(1, D)
    b2s = b2_rel_c.reshape(1, D)

    p_ii, p_c, p_cb = _seg3(x_drug, x_se, si_ii, di_ii, si_c, di_c,
                            si_cb, di_cb)
    d1, s1 = _dense(p_ii, p_cb, p_c, x_drug, x_se,
                    W1_rel_ii, W1_rel_cb, W1_rel_c, w1rd, W1_root_c,
                    b1d, b1s)
    q_ii, q_c, q_cb = _seg3(d1, s1, si_ii, di_ii, si_c, di_c, si_cb, di_cb)
    d2, s2 = _dense(q_ii, q_cb, q_c, d1, s1,
                    W2_rel_ii, W2_rel_cb, W2_rel_c, w2rd, W2_root_c,
                    b2d, b2s)
    return _bn(d2, s2, bn_gamma.reshape(1, D), bn_beta.reshape(1, D))


# speedup vs baseline: 9.2085x; 1.2869x over previous
import functools

import jax
import jax.numpy as jnp
from jax import lax
from jax.experimental import pallas as pl
from jax.experimental.pallas import tpu as pltpu
from jax.experimental.pallas import tpu_sc as plsc

ND = 10000
D = 128
E = 320000
NC = 2
NS = 16
NW = NC * NS
K = 80
CH_TOT = E // K
CHW = CH_TOT // NW
RPT = 624
TAIL = ND - NS * RPT
ZR = 48

_f32 = jnp.float32


def _seg3_body(xa, xb, si_ii, di_ii, si_c, di_c, si_cb, di_cb,
               out_ii, out_c, out_cb,
               acc, zbuf, sidx0, didx0, sidx1, didx1, sidx2, didx2,
               rows0, rows1, rows2,
               smi0, smd0, smi1, smd1, smi2, smd2, smg0, smg1, smg2):
    c = lax.axis_index("c")
    s = lax.axis_index("s")
    wid = c * NS + s
    sbuf = (sidx0, sidx1, sidx2)
    dbuf = (didx0, didx1, didx2)
    rbuf = (rows0, rows1, rows2)
    smi = (smi0, smi1, smi2)
    smd = (smd0, smd1, smd2)
    smg = (smg0, smg1, smg2)

    z16 = jnp.zeros((16,), _f32)

    def zrow(r, carry):
        for j in range(D // 16):
            zbuf[r, pl.ds(j * 16, 16)] = z16
        return carry

    lax.fori_loop(0, ZR, zrow, 0)

    for x_hbm, si_hbm, di_hbm, out_hbm in (
        (xa, si_ii, di_ii, out_ii),
        (xa, si_c, di_c, out_c),
        (xb, si_cb, di_cb, out_cb),
    ):
        for j in range(RPT // ZR):
            pltpu.sync_copy(zbuf, acc.at[pl.ds(s * RPT + j * ZR, ZR)])

        @pl.when(s == NS - 1)
        def _():
            pltpu.sync_copy(zbuf.at[pl.ds(0, TAIL)],
                            acc.at[pl.ds(NS * RPT, TAIL)])

        plsc.subcore_barrier()

        cbase = wid * CHW

        def idx_start(ch, b):
            off = pl.multiple_of(jnp.minimum(ch, CH_TOT - 1) * K, 8)
            pltpu.async_copy(si_hbm.at[pl.ds(off, K)], sbuf[b], smi[b])
            pltpu.async_copy(di_hbm.at[pl.ds(off, K)], dbuf[b], smd[b])

        def idx_wait(b):
            pltpu.make_async_copy(si_hbm.at[pl.ds(0, K)], sbuf[b],
                                  smi[b]).wait()
            pltpu.make_async_copy(di_hbm.at[pl.ds(0, K)], dbuf[b],
                                  smd[b]).wait()

        def gather_start(b):
            pltpu.async_copy(x_hbm.at[sbuf[b]], rbuf[b], smg[b])

        def gather_wait(b):
            pltpu.make_async_copy(x_hbm.at[pl.ds(0, K)], rbuf[b],
                                  smg[b]).wait()

        def scat(b):
            pltpu.sync_copy(rbuf[b], acc.at[dbuf[b]], add=True)

        idx_start(cbase, 0)
        idx_start(cbase + 1, 1)
        idx_start(cbase + 2, 2)
        idx_wait(0)
        gather_start(0)
        idx_wait(1)
        gather_start(1)

        def tri(j, carry):
            i = 3 * j
            for b in range(3):
                gather_wait(b)
                scat(b)
                idx_wait((b + 2) % 3)
                gather_start((b + 2) % 3)
                idx_start(cbase + i + b + 3, b)
            return carry

        lax.fori_loop(0, (CHW - 2) // 3, tri, 0)
        gather_wait(0)
        scat(0)
        gather_wait(1)
        scat(1)
        idx_wait(2)
        plsc.subcore_barrier()

        pltpu.sync_copy(acc.at[pl.ds(s * RPT, RPT)],
                        out_hbm.at[c, pl.ds(s * RPT, RPT)])

        @pl.when(s == NS - 1)
        def _():
            pltpu.sync_copy(acc.at[pl.ds(NS * RPT, TAIL)],
                            out_hbm.at[c, pl.ds(NS * RPT, TAIL)])


def _seg3(xa, xb, si_ii, di_ii, si_c, di_c, si_cb, di_cb):
    mesh = plsc.VectorSubcoreMesh(core_axis_name="c", subcore_axis_name="s")
    f = functools.partial(
        pl.kernel,
        mesh=mesh,
        out_type=[jax.ShapeDtypeStruct((NC, ND, D), _f32)] * 3,
        scratch_types=[
            pltpu.VMEM_SHARED((ND, D), _f32),
            pltpu.VMEM((ZR, D), _f32),
            pltpu.VMEM((K,), jnp.int32),
            pltpu.VMEM((K,), jnp.int32),
            pltpu.VMEM((K,), jnp.int32),
            pltpu.VMEM((K,), jnp.int32),
            pltpu.VMEM((K,), jnp.int32),
            pltpu.VMEM((K,), jnp.int32),
            pltpu.VMEM((K, D), _f32),
            pltpu.VMEM((K, D), _f32),
            pltpu.VMEM((K, D), _f32),
        ] + [pltpu.SemaphoreType.DMA] * 9,
    )(_seg3_body)
    return f(xa, xb, si_ii, di_ii, si_c, di_c, si_cb, di_cb)


_RB = 1000


def _dense_body(aii, acb, ac, xd, xs, wri, wrcb, wrc, wrd, wrs, bd, bs,
                d_o, s_o):
    agg_ii = aii[0] + aii[1]
    agg_cb = acb[0] + acb[1]
    agg_c = ac[0] + ac[1]
    d = (jnp.dot(agg_ii, wri[...], preferred_element_type=_f32)
         + jnp.dot(agg_cb, wrcb[...], preferred_element_type=_f32)
         + jnp.dot(xd[...], wrd[...], preferred_element_type=_f32)
         + bd[...])
    s = (jnp.dot(agg_c, wrc[...], preferred_element_type=_f32)
         + jnp.dot(xs[...], wrs[...], preferred_element_type=_f32)
         + bs[...])
    d_o[...] = jnp.where(d >= 0, d, 0.01 * d)
    s_o[...] = jnp.where(s >= 0, s, 0.01 * s)


def _dense(p_ii, p_cb, p_c, xd, xs, wri, wrcb, wrc, wrd, wrs, bd, bs):
    n = ND // _RB
    part = pl.BlockSpec((2, _RB, D), lambda i: (0, i, 0))
    row = pl.BlockSpec((_RB, D), lambda i: (i, 0))
    mat = pl.BlockSpec((D, D), lambda i: (0, 0))
    vec = pl.BlockSpec((1, D), lambda i: (0, 0))
    return pl.pallas_call(
        _dense_body,
        grid=(n,),
        in_specs=[part, part, part, row, row, mat, mat, mat, mat, mat,
                  vec, vec],
        out_specs=[row, row],
        out_shape=[jax.ShapeDtypeStruct((ND, D), _f32)] * 2,
    )(p_ii, p_cb, p_c, xd, xs, wri, wrcb, wrc, wrd, wrs, bd, bs)


def _bn_body(d2, s2, g, b, d_o, s_o):
    gv = g[...]
    bv = b[...]
    for x, o in ((d2, d_o), (s2, s_o)):
        xv = x[...]
        m = jnp.mean(xv, axis=0, keepdims=True)
        cv = xv - m
        v = jnp.mean(cv * cv, axis=0, keepdims=True)
        o[...] = cv * lax.rsqrt(v + 1e-5) * gv + bv


def _bn(d2, s2, gamma, beta):
    full = pl.BlockSpec((ND, D), lambda: (0, 0))
    vec = pl.BlockSpec((1, D), lambda: (0, 0))
    return pl.pallas_call(
        _bn_body,
        in_specs=[full, full, vec, vec],
        out_specs=[full, full],
        out_shape=[jax.ShapeDtypeStruct((ND, D), _f32)] * 2,
    )(d2, s2, gamma, beta)


def kernel(x_drug, x_se, ei_interacts, ei_causes, ei_caused_by,
           W1_rel_ii, b1_rel_ii, W1_root_ii, W1_rel_c, b1_rel_c, W1_root_c,
           W1_rel_cb, b1_rel_cb, W1_root_cb,
           W2_rel_ii, b2_rel_ii, W2_root_ii, W2_rel_c, b2_rel_c, W2_root_c,
           W2_rel_cb, b2_rel_cb, W2_root_cb, bn_gamma, bn_beta):
    si_ii, di_ii = ei_interacts[0], ei_interacts[1]
    si_c, di_c = ei_causes[0], ei_causes[1]
    si_cb, di_cb = ei_caused_by[0], ei_caused_by[1]

    w1rd = W1_root_ii + W1_root_cb
    b1d = (b1_rel_ii + b1_rel_cb).reshape(1, D)
    b1s = b1_rel_c.reshape(1, D)
    w2rd = W2_root_ii + W2_root_cb
    b2d = (b2_rel_ii + b2_rel_cb).reshape(1, D)
    b2s = b2_rel_c.reshape(1, D)

    p_ii, p_c, p_cb = _seg3(x_drug, x_se, si_ii, di_ii, si_c, di_c,
                            si_cb, di_cb)
    d1, s1 = _dense(p_ii, p_cb, p_c, x_drug, x_se,
                    W1_rel_ii, W1_rel_cb, W1_rel_c, w1rd, W1_root_c,
                    b1d, b1s)
    q_ii, q_c, q_cb = _seg3(d1, s1, si_ii, di_ii, si_c, di_c, si_cb, di_cb)
    d2, s2 = _dense(q_ii, q_cb, q_c, d1, s1,
                    W2_rel_ii, W2_rel_cb, W2_rel_c, w2rd, W2_root_c,
                    b2d, b2s)
    return _bn(d2, s2, bn_gamma.reshape(1, D), bn_beta.reshape(1, D))
